# baseline (speedup 1.0000x reference)
import functools

import jax
import jax.numpy as jnp
from jax import lax
from jax.experimental import pallas as pl
from jax.experimental.pallas import tpu as pltpu
from jax.experimental.pallas import tpu_sc as plsc

_GAMMA = 12.0
_B = 16384
_D = 128
_NUM_REL = 1000
_LANES = 16
_CHUNK = 32
_NSLOTS_H = 5
_NSLOTS_T = 4


def _sc_geometry():
    try:
        info = plsc.get_sparse_core_info()
        return info.num_cores, info.num_subcores
    except Exception:
        return 2, 16


def _kge_body(nc, bpw, ent_hbm, rel_hbm, sample_hbm, out_hbm, *scratch):
    nb = _NSLOTS_H + _NSLOTS_T
    sview, hidx_v, ridx_v, tidx_v = scratch[:4]
    hbufs = scratch[4:4 + _NSLOTS_H]
    tbufs = scratch[4 + _NSLOTS_H:4 + nb]
    out_v, rel_sh = scratch[4 + nb:6 + nb]
    sems = scratch[6 + nb:]
    hsems = sems[:_NSLOTS_H]
    tsems = sems[_NSLOTS_H:nb]
    sem_stage = sems[nb]

    wid = lax.axis_index("s") * nc + lax.axis_index("c")
    base = wid * bpw
    lane = lax.iota(jnp.int32, _LANES)
    sid = lax.axis_index("s")

    @pl.when(sid == 0)
    def _stage():
        pltpu.async_copy(rel_hbm, rel_sh, sem_stage)

    pltpu.sync_copy(sample_hbm.at[pl.ds(base, bpw)], sview)

    def split_body(i, carry):
        rows = i * _LANES + lane
        for col, dst in ((0, hidx_v), (1, ridx_v), (2, tidx_v)):
            cols = jnp.full((_LANES,), col, jnp.int32)
            dst[pl.ds(i * _LANES, _LANES)] = (
                plsc.load_gather(sview, [rows, cols]))
        return carry

    lax.fori_loop(0, bpw // _LANES, split_body, 0)

    nchunks = bpw // _CHUNK

    def issue_h(c):
        off = c * _CHUNK
        s = c % _NSLOTS_H
        return pltpu.async_copy(ent_hbm.at[hidx_v.at[pl.ds(off, _CHUNK)]],
                                hbufs[s], hsems[s])

    def issue_t(c):
        off = c * _CHUNK
        s = c % _NSLOTS_T
        return pltpu.async_copy(ent_hbm.at[tidx_v.at[pl.ds(off, _CHUNK)]],
                                tbufs[s], tsems[s])

    def issue_radd(c, cp_h):
        cp_h.wait()
        off = c * _CHUNK
        s = c % _NSLOTS_H
        return pltpu.async_copy(rel_sh.at[ridx_v.at[pl.ds(off, _CHUNK)]],
                                hbufs[s], hsems[s], add=True)

    def compute(c):
        off = c * _CHUNK
        hb = hbufs[c % _NSLOTS_H]
        tb = tbufs[c % _NSLOTS_T]

        def group_body(g, carry):
            gbase = g * _LANES

            def quad_body(q, scores):
                for u in range(4):
                    j = q * 4 + u
                    i = gbase + j
                    accs = [jnp.zeros((_LANES,), jnp.float32)
                            for _ in range(4)]
                    for k in range(_D // _LANES):
                        hr = hb[i, pl.ds(k * _LANES, _LANES)]
                        t = tb[i, pl.ds(k * _LANES, _LANES)]
                        accs[k % 4] = accs[k % 4] + jnp.abs(hr - t)
                    while len(accs) > 1:
                        accs = [a + b for a, b in zip(accs[::2], accs[1::2])]
                    s = jnp.sum(accs[0])
                    scores = jnp.where(lane == j, _GAMMA - s, scores)
                return scores

            scores = lax.fori_loop(0, _LANES // 4, quad_body,
                                   jnp.zeros((_LANES,), jnp.float32))
            out_v[pl.ds(off + gbase, _LANES)] = scores
            return carry

        lax.fori_loop(0, _CHUNK // _LANES, group_body, 0)

    pend_h, pend_r, pend_t = {}, {}, {}
    for c0 in range(min(_NSLOTS_H - 1, nchunks)):
        pend_h[c0] = issue_h(c0)
    for c0 in range(min(_NSLOTS_T - 1, nchunks)):
        pend_t[c0] = issue_t(c0)

    @pl.when(sid == 0)
    def _stage_wait():
        pltpu.make_async_copy(rel_hbm, rel_sh, sem_stage).wait()

    plsc.subcore_barrier()
    pend_r[0] = issue_radd(0, pend_h.pop(0))
    for c in range(nchunks):
        if c + _NSLOTS_H - 1 < nchunks:
            pend_h[c + _NSLOTS_H - 1] = issue_h(c + _NSLOTS_H - 1)
        if c + _NSLOTS_T - 1 < nchunks:
            pend_t[c + _NSLOTS_T - 1] = issue_t(c + _NSLOTS_T - 1)
        if c + 1 < nchunks:
            pend_r[c + 1] = issue_radd(c + 1, pend_h.pop(c + 1))
        pend_r.pop(c).wait()
        pend_t.pop(c).wait()
        compute(c)

    pltpu.sync_copy(out_v, out_hbm.at[pl.ds(base, bpw)])


def kernel(entity_embedding, relation_embedding, sample):
    nc, ns = _sc_geometry()
    nw = nc * ns
    bpw = _B // nw

    mesh = plsc.VectorSubcoreMesh(core_axis_name="c", subcore_axis_name="s")
    scratch = [
        pltpu.VMEM((bpw, 3), jnp.int32),
        pltpu.VMEM((bpw,), jnp.int32),
        pltpu.VMEM((bpw,), jnp.int32),
        pltpu.VMEM((bpw,), jnp.int32),
    ]
    scratch += [pltpu.VMEM((_CHUNK, _D), jnp.float32)
                for _ in range(_NSLOTS_H)]
    scratch += [pltpu.VMEM((_CHUNK, _D), jnp.float32)
                for _ in range(_NSLOTS_T)]
    scratch += [
        pltpu.VMEM((bpw,), jnp.float32),
        pltpu.VMEM_SHARED((_NUM_REL, _D), jnp.float32),
    ]
    scratch += [pltpu.SemaphoreType.DMA
                for _ in range(_NSLOTS_H + _NSLOTS_T + 1)]

    kge = functools.partial(
        pl.kernel,
        mesh=mesh,
        compiler_params=pltpu.CompilerParams(needs_layout_passes=False),
        out_type=jax.ShapeDtypeStruct((_B,), jnp.float32),
        scratch_types=scratch,
    )(functools.partial(_kge_body, nc, bpw))
    scores = kge(entity_embedding, relation_embedding, sample)
    return scores[:, None]


# speedup vs baseline: 1.0338x; 1.0338x over previous
import functools

import jax
import jax.numpy as jnp
from jax import lax
from jax.experimental import pallas as pl
from jax.experimental.pallas import tpu as pltpu
from jax.experimental.pallas import tpu_sc as plsc

_GAMMA = 12.0
_B = 16384
_D = 128
_NUM_REL = 1000
_LANES = 16
_CHUNK = 64
_NSLOTS = 3


def _sc_geometry():
    try:
        info = plsc.get_sparse_core_info()
        return info.num_cores, info.num_subcores
    except Exception:
        return 2, 16


def _kge_body(nc, bpw, ent_hbm, rel_hbm, sample_hbm, out_hbm, *scratch):
    sview, hidx_v, ridx_v, tidx_v = scratch[:4]
    hbufs = scratch[4:4 + _NSLOTS]
    tbufs = scratch[4 + _NSLOTS:4 + 2 * _NSLOTS]
    out_v, rel_sh = scratch[4 + 2 * _NSLOTS:6 + 2 * _NSLOTS]
    sems = scratch[6 + 2 * _NSLOTS:]
    hsems = sems[:_NSLOTS]
    tsems = sems[_NSLOTS:2 * _NSLOTS]
    sem_stage = sems[2 * _NSLOTS]

    wid = lax.axis_index("s") * nc + lax.axis_index("c")
    base = wid * bpw
    lane = lax.iota(jnp.int32, _LANES)
    sid = lax.axis_index("s")

    @pl.when(sid == 0)
    def _stage():
        pltpu.async_copy(rel_hbm, rel_sh, sem_stage)

    pltpu.sync_copy(sample_hbm.at[pl.ds(base, bpw)], sview)

    def split_body(i, carry):
        rows = i * _LANES + lane
        for col, dst in ((0, hidx_v), (1, ridx_v), (2, tidx_v)):
            cols = jnp.full((_LANES,), col, jnp.int32)
            dst[pl.ds(i * _LANES, _LANES)] = (
                plsc.load_gather(sview, [rows, cols]))
        return carry

    head_groups = (_NSLOTS - 1) * _CHUNK // _LANES
    lax.fori_loop(0, head_groups, split_body, 0)

    nchunks = bpw // _CHUNK

    def issue_ht(c):
        off = c * _CHUNK
        s = c % _NSLOTS
        cp_h = pltpu.async_copy(ent_hbm.at[hidx_v.at[pl.ds(off, _CHUNK)]],
                                hbufs[s], hsems[s])
        cp_t = pltpu.async_copy(ent_hbm.at[tidx_v.at[pl.ds(off, _CHUNK)]],
                                tbufs[s], tsems[s])
        return cp_h, cp_t

    def issue_radd(c, cp_h):
        cp_h.wait()
        off = c * _CHUNK
        s = c % _NSLOTS
        return pltpu.async_copy(rel_sh.at[ridx_v.at[pl.ds(off, _CHUNK)]],
                                hbufs[s], hsems[s], add=True)

    def compute(c):
        off = c * _CHUNK
        hb = hbufs[c % _NSLOTS]
        tb = tbufs[c % _NSLOTS]

        def group_body(g, carry):
            gbase = g * _LANES

            def quad_body(q, scores):
                for u in range(4):
                    j = q * 4 + u
                    i = gbase + j
                    accs = [jnp.zeros((_LANES,), jnp.float32)
                            for _ in range(4)]
                    for k in range(_D // _LANES):
                        hr = hb[i, pl.ds(k * _LANES, _LANES)]
                        t = tb[i, pl.ds(k * _LANES, _LANES)]
                        accs[k % 4] = accs[k % 4] + jnp.abs(hr - t)
                    while len(accs) > 1:
                        accs = [a + b for a, b in zip(accs[::2], accs[1::2])]
                    s = jnp.sum(accs[0])
                    scores = jnp.where(lane == j, _GAMMA - s, scores)
                return scores

            scores = lax.fori_loop(0, _LANES // 4, quad_body,
                                   jnp.zeros((_LANES,), jnp.float32))
            out_v[pl.ds(off + gbase, _LANES)] = scores
            return carry

        lax.fori_loop(0, _CHUNK // _LANES, group_body, 0)

    pend_h, pend_r, pend_t = {}, {}, {}
    for c0 in range(min(_NSLOTS - 1, nchunks)):
        cp_h, cp_t = issue_ht(c0)
        pend_h[c0] = cp_h
        pend_t[c0] = cp_t

    lax.fori_loop(head_groups, bpw // _LANES, split_body, 0)

    @pl.when(sid == 0)
    def _stage_wait():
        pltpu.make_async_copy(rel_hbm, rel_sh, sem_stage).wait()

    plsc.subcore_barrier()
    pend_r[0] = issue_radd(0, pend_h.pop(0))
    for c in range(nchunks):
        if c + _NSLOTS - 1 < nchunks:
            cp_h, cp_t = issue_ht(c + _NSLOTS - 1)
            pend_h[c + _NSLOTS - 1] = cp_h
            pend_t[c + _NSLOTS - 1] = cp_t
        if c + 1 < nchunks:
            pend_r[c + 1] = issue_radd(c + 1, pend_h.pop(c + 1))
        pend_r.pop(c).wait()
        pend_t.pop(c).wait()
        compute(c)

    pltpu.sync_copy(out_v, out_hbm.at[pl.ds(base, bpw)])


def kernel(entity_embedding, relation_embedding, sample):
    nc, ns = _sc_geometry()
    nw = nc * ns
    bpw = _B // nw

    mesh = plsc.VectorSubcoreMesh(core_axis_name="c", subcore_axis_name="s")
    scratch = [
        pltpu.VMEM((bpw, 3), jnp.int32),
        pltpu.VMEM((bpw,), jnp.int32),
        pltpu.VMEM((bpw,), jnp.int32),
        pltpu.VMEM((bpw,), jnp.int32),
    ]
    scratch += [pltpu.VMEM((_CHUNK, _D), jnp.float32)
                for _ in range(_NSLOTS)]
    scratch += [pltpu.VMEM((_CHUNK, _D), jnp.float32)
                for _ in range(_NSLOTS)]
    scratch += [
        pltpu.VMEM((bpw,), jnp.float32),
        pltpu.VMEM_SHARED((_NUM_REL, _D), jnp.float32),
    ]
    scratch += [pltpu.SemaphoreType.DMA for _ in range(2 * _NSLOTS + 1)]

    kge = functools.partial(
        pl.kernel,
        mesh=mesh,
        compiler_params=pltpu.CompilerParams(needs_layout_passes=False),
        out_type=jax.ShapeDtypeStruct((_B,), jnp.float32),
        scratch_types=scratch,
    )(functools.partial(_kge_body, nc, bpw))
    scores = kge(entity_embedding, relation_embedding, sample)
    return scores[:, None]


# speedup vs baseline: 1.0344x; 1.0005x over previous
import functools

import jax
import jax.numpy as jnp
from jax import lax
from jax.experimental import pallas as pl
from jax.experimental.pallas import tpu as pltpu
from jax.experimental.pallas import tpu_sc as plsc

_GAMMA = 12.0
_B = 16384
_D = 128
_NUM_REL = 1000
_LANES = 16
_CHUNK = 64
_NSLOTS = 3


def _sc_geometry():
    try:
        info = plsc.get_sparse_core_info()
        return info.num_cores, info.num_subcores
    except Exception:
        return 2, 16


def _kge_body(nc, bpw, ent_hbm, rel_hbm, sample_hbm, out_hbm, *scratch):
    sview, hidx_v, ridx_v, tidx_v = scratch[:4]
    hbufs = scratch[4:4 + _NSLOTS]
    tbufs = scratch[4 + _NSLOTS:4 + 2 * _NSLOTS]
    out_v, rel_sh = scratch[4 + 2 * _NSLOTS:6 + 2 * _NSLOTS]
    sems = scratch[6 + 2 * _NSLOTS:]
    hsems = sems[:_NSLOTS]
    tsems = sems[_NSLOTS:2 * _NSLOTS]
    sem_stage = sems[2 * _NSLOTS]

    wid = lax.axis_index("s") * nc + lax.axis_index("c")
    base = wid * bpw
    lane = lax.iota(jnp.int32, _LANES)
    sid = lax.axis_index("s")

    @pl.when(sid == 0)
    def _stage():
        pltpu.async_copy(rel_hbm, rel_sh, sem_stage)

    pltpu.sync_copy(sample_hbm.at[pl.ds(base, bpw)], sview)

    def split_body(i, carry):
        rows = i * _LANES + lane
        for col, dst in ((0, hidx_v), (1, ridx_v), (2, tidx_v)):
            cols = jnp.full((_LANES,), col, jnp.int32)
            dst[pl.ds(i * _LANES, _LANES)] = (
                plsc.load_gather(sview, [rows, cols]))
        return carry

    head_groups = (_NSLOTS - 1) * _CHUNK // _LANES
    lax.fori_loop(0, head_groups, split_body, 0)

    nchunks = bpw // _CHUNK

    half = _CHUNK // 2

    def issue_ht(c):
        off = c * _CHUNK
        s = c % _NSLOTS
        cps = []
        for p in range(2):
            cps.append(pltpu.async_copy(
                ent_hbm.at[hidx_v.at[pl.ds(off + p * half, half)]],
                hbufs[s].at[pl.ds(p * half, half)], hsems[s]))
        for p in range(2):
            cps.append(pltpu.async_copy(
                ent_hbm.at[tidx_v.at[pl.ds(off + p * half, half)]],
                tbufs[s].at[pl.ds(p * half, half)], tsems[s]))
        return tuple(cps)

    def issue_radd(c, cp_hs):
        for cp in cp_hs:
            cp.wait()
        off = c * _CHUNK
        s = c % _NSLOTS
        return pltpu.async_copy(rel_sh.at[ridx_v.at[pl.ds(off, _CHUNK)]],
                                hbufs[s], hsems[s], add=True)

    def compute(c):
        off = c * _CHUNK
        hb = hbufs[c % _NSLOTS]
        tb = tbufs[c % _NSLOTS]

        def group_body(g, carry):
            gbase = g * _LANES

            def quad_body(q, scores):
                for u in range(4):
                    j = q * 4 + u
                    i = gbase + j
                    accs = [jnp.zeros((_LANES,), jnp.float32)
                            for _ in range(4)]
                    for k in range(_D // _LANES):
                        hr = hb[i, pl.ds(k * _LANES, _LANES)]
                        t = tb[i, pl.ds(k * _LANES, _LANES)]
                        accs[k % 4] = accs[k % 4] + jnp.abs(hr - t)
                    while len(accs) > 1:
                        accs = [a + b for a, b in zip(accs[::2], accs[1::2])]
                    s = jnp.sum(accs[0])
                    scores = jnp.where(lane == j, _GAMMA - s, scores)
                return scores

            scores = lax.fori_loop(0, _LANES // 4, quad_body,
                                   jnp.zeros((_LANES,), jnp.float32))
            out_v[pl.ds(off + gbase, _LANES)] = scores
            return carry

        lax.fori_loop(0, _CHUNK // _LANES, group_body, 0)

    pend_h, pend_r, pend_t = {}, {}, {}
    for c0 in range(min(_NSLOTS - 1, nchunks)):
        cps = issue_ht(c0)
        pend_h[c0] = cps[:2]
        pend_t[c0] = cps[2:]

    lax.fori_loop(head_groups, bpw // _LANES, split_body, 0)

    @pl.when(sid == 0)
    def _stage_wait():
        pltpu.make_async_copy(rel_hbm, rel_sh, sem_stage).wait()

    plsc.subcore_barrier()
    pend_r[0] = issue_radd(0, pend_h.pop(0))
    for c in range(nchunks):
        if c + _NSLOTS - 1 < nchunks:
            cps = issue_ht(c + _NSLOTS - 1)
            pend_h[c + _NSLOTS - 1] = cps[:2]
            pend_t[c + _NSLOTS - 1] = cps[2:]
        if c + 1 < nchunks:
            pend_r[c + 1] = issue_radd(c + 1, pend_h.pop(c + 1))
        pend_r.pop(c).wait()
        for cp in pend_t.pop(c):
            cp.wait()
        compute(c)

    pltpu.sync_copy(out_v, out_hbm.at[pl.ds(base, bpw)])


def kernel(entity_embedding, relation_embedding, sample):
    nc, ns = _sc_geometry()
    nw = nc * ns
    bpw = _B // nw

    mesh = plsc.VectorSubcoreMesh(core_axis_name="c", subcore_axis_name="s")
    scratch = [
        pltpu.VMEM((bpw, 3), jnp.int32),
        pltpu.VMEM((bpw,), jnp.int32),
        pltpu.VMEM((bpw,), jnp.int32),
        pltpu.VMEM((bpw,), jnp.int32),
    ]
    scratch += [pltpu.VMEM((_CHUNK, _D), jnp.float32)
                for _ in range(_NSLOTS)]
    scratch += [pltpu.VMEM((_CHUNK, _D), jnp.float32)
                for _ in range(_NSLOTS)]
    scratch += [
        pltpu.VMEM((bpw,), jnp.float32),
        pltpu.VMEM_SHARED((_NUM_REL, _D), jnp.float32),
    ]
    scratch += [pltpu.SemaphoreType.DMA for _ in range(2 * _NSLOTS + 1)]

    kge = functools.partial(
        pl.kernel,
        mesh=mesh,
        compiler_params=pltpu.CompilerParams(needs_layout_passes=False),
        out_type=jax.ShapeDtypeStruct((_B,), jnp.float32),
        scratch_types=scratch,
    )(functools.partial(_kge_body, nc, bpw))
    scores = kge(entity_embedding, relation_embedding, sample)
    return scores[:, None]
